# Initial kernel scaffold; baseline (speedup 1.0000x reference)
import jax
import jax.numpy as jnp
from jax.experimental import pallas as pl


def kernel(indices, table):
    raise NotImplementedError("write your pallas kernel here")


# speedup vs baseline: 4.2146x; 4.2146x over previous
import functools

import jax
import jax.numpy as jnp
from jax import lax
from jax.experimental import pallas as pl
from jax.experimental.pallas import tpu as pltpu
from jax.experimental.pallas import tpu_sc as plsc

_NUM_CORES = 2
_NUM_SUBCORES = 16
_NW = _NUM_CORES * _NUM_SUBCORES


@functools.partial(jax.jit, static_argnames=("chunk",))
def _gather_sc(idx_flat, table, chunk):
    n = idx_flat.shape[0]
    d = table.shape[1]
    b_per_w = n // _NW
    n_chunks = b_per_w // chunk

    mesh = plsc.VectorSubcoreMesh(
        core_axis_name="c", subcore_axis_name="s",
        num_cores=_NUM_CORES, num_subcores=_NUM_SUBCORES,
    )

    @functools.partial(
        pl.kernel,
        mesh=mesh,
        compiler_params=pltpu.CompilerParams(use_tc_tiling_on_sc=False),
        out_type=jax.ShapeDtypeStruct((n, d), jnp.float32),
        scratch_types=[
            pltpu.VMEM((chunk,), jnp.int32),
            pltpu.VMEM((chunk, d), jnp.float32),
            pltpu.SemaphoreType.DMA,
        ],
    )
    def k(idx_hbm, table_hbm, out_hbm, idx_v, rows_v, sem):
        wid = lax.axis_index("s") * _NUM_CORES + lax.axis_index("c")
        base = wid * b_per_w

        def body(c, carry):
            off = base + c * chunk
            pltpu.sync_copy(idx_hbm.at[pl.ds(off, chunk)], idx_v)
            pltpu.async_copy(table_hbm.at[idx_v], rows_v, sem).wait()
            pltpu.sync_copy(rows_v, out_hbm.at[pl.ds(off, chunk)])
            return carry

        lax.fori_loop(0, n_chunks, body, 0)

    return k(idx_flat, table)


def kernel(indices, table):
    b, h = indices.shape
    idx_flat = indices.reshape(b * h).astype(jnp.int32)
    out = _gather_sc(idx_flat, table, chunk=1600)
    return out.reshape(b, h, table.shape[1])


# speedup vs baseline: 4.2197x; 1.0012x over previous
import functools

import jax
import jax.numpy as jnp
from jax import lax
from jax.experimental import pallas as pl
from jax.experimental.pallas import tpu as pltpu
from jax.experimental.pallas import tpu_sc as plsc

_NUM_CORES = 2
_NUM_SUBCORES = 16
_NW = _NUM_CORES * _NUM_SUBCORES
_NBUF = 2


@functools.partial(jax.jit, static_argnames=("chunk",))
def _gather_sc(idx_flat, table, chunk):
    n = idx_flat.shape[0]
    d = table.shape[1]
    b_per_w = n // _NW
    n_chunks = b_per_w // chunk
    assert n_chunks % _NBUF == 0 and n_chunks >= 2 * _NBUF

    mesh = plsc.VectorSubcoreMesh(
        core_axis_name="c", subcore_axis_name="s",
        num_cores=_NUM_CORES, num_subcores=_NUM_SUBCORES,
    )

    @functools.partial(
        pl.kernel,
        mesh=mesh,
        compiler_params=pltpu.CompilerParams(use_tc_tiling_on_sc=False),
        out_type=jax.ShapeDtypeStruct((n, d), jnp.float32),
        scratch_types=[
            pltpu.VMEM((_NBUF, chunk), jnp.int32),
            pltpu.VMEM((_NBUF, chunk, d), jnp.float32),
            pltpu.SemaphoreType.DMA((_NBUF,)),
            pltpu.SemaphoreType.DMA((_NBUF,)),
        ],
    )
    def k(idx_hbm, table_hbm, out_hbm, idx_v, rows_v, gsem, osem):
        wid = lax.axis_index("s") * _NUM_CORES + lax.axis_index("c")
        base = wid * b_per_w

        def step(cc, b, wait_out):
            off = base + cc * chunk
            if wait_out:
                pltpu.make_async_copy(
                    rows_v.at[b], out_hbm.at[pl.ds(off, chunk)], osem.at[b]
                ).wait()
            pltpu.sync_copy(idx_hbm.at[pl.ds(off, chunk)], idx_v.at[b])
            pltpu.async_copy(
                table_hbm.at[idx_v.at[b]], rows_v.at[b], gsem.at[b]
            ).wait()
            pltpu.async_copy(
                rows_v.at[b], out_hbm.at[pl.ds(off, chunk)], osem.at[b]
            )

        for b in range(_NBUF):
            step(jnp.int32(b), b, wait_out=False)

        def body(r, carry):
            c0 = _NBUF + r * _NBUF
            for b in range(_NBUF):
                step(c0 + b, b, wait_out=True)
            return carry

        lax.fori_loop(0, n_chunks // _NBUF - 1, body, 0)

        for b in range(_NBUF):
            off = base + (n_chunks - _NBUF + b) * chunk
            pltpu.make_async_copy(
                rows_v.at[b], out_hbm.at[pl.ds(off, chunk)], osem.at[b]
            ).wait()

    return k(idx_flat, table)


def kernel(indices, table):
    b, h = indices.shape
    idx_flat = indices.reshape(b * h).astype(jnp.int32)
    out = _gather_sc(idx_flat, table, chunk=800)
    return out.reshape(b, h, table.shape[1])
